# Initial kernel scaffold; baseline (speedup 1.0000x reference)
import jax
import jax.numpy as jnp
from jax.experimental import pallas as pl


def kernel(x, edge_index, batch, epoch, W1, b1, p1, W2, b2, p2, W3, b3, Wm1, bm1, Wm2, bm2):
    raise NotImplementedError("write your pallas kernel here")


# speedup vs baseline: 3.8702x; 3.8702x over previous
import functools
import math

import jax
import jax.numpy as jnp
from jax import lax
from jax.experimental import pallas as pl
from jax.experimental.pallas import tpu as pltpu
from jax.experimental.pallas import tpu_sc as plsc

N0 = 10000
NUM_GRAPHS = 64
E0 = 320000
K1 = int(math.ceil(0.5 * N0))
K2 = int(math.ceil(0.5 * K1))

NC = 2
NS = 16
NW = NC * NS
EP = 327680
BM = 256

NEG = -3.0e38


def _pad_rows(n):
    return ((n + BM - 1) // BM) * BM


def _pad_acc(n):
    return _pad_rows(n) + 256


def _chunk_for(d):
    return {128: 128, 256: 128, 512: 64}[d]


def _sc_mesh():
    return plsc.VectorSubcoreMesh(core_axis_name="c", subcore_axis_name="s",
                                  num_cores=NC, num_subcores=NS)


CF = 2048
NCF = EP // (NC * CF)
CAP = 167936
CBN = CF + 128


def sc_partition(sh, dh, np_acc, npad):
    rows_pt = np_acc // NS
    locpad = rows_pt - 16

    @functools.partial(
        pl.kernel,
        mesh=_sc_mesh(),
        compiler_params=pltpu.CompilerParams(needs_layout_passes=False),
        out_type=(jax.ShapeDtypeStruct((NW * rows_pt * 16,), jnp.float32),
                  jax.ShapeDtypeStruct((NW * CAP,), jnp.int32),
                  jax.ShapeDtypeStruct((NW * CAP,), jnp.int32),
                  jax.ShapeDtypeStruct((NW * 8,), jnp.int32)),
        scratch_types=[
            pltpu.VMEM((CF,), jnp.int32),
            pltpu.VMEM((CF,), jnp.int32),
            pltpu.VMEM((CBN,), jnp.int32),
            pltpu.VMEM((CBN,), jnp.int32),
            pltpu.VMEM((rows_pt * 16,), jnp.float32),
            pltpu.VMEM((16,), jnp.int32),
        ],
    )
    def k(sh_hbm, dh_hbm, deg_hbm, sl_hbm, ll_hbm, cnt_hbm,
          sv, dv, cbs, cbl, degv, cntv):
        c = lax.axis_index("c")
        s = lax.axis_index("s")
        w = c * NS + s
        base = w * CAP

        z16 = jnp.zeros((16,), jnp.float32)
        one16 = (lax.iota(jnp.int32, 16) == 0).astype(jnp.float32)
        iota16 = lax.iota(jnp.int32, 16)
        lp16 = jnp.full((16,), locpad, jnp.int32)
        z16i = jnp.zeros((16,), jnp.int32)

        def zdeg(i, _):
            degv[pl.ds(i * 16, 16)] = z16
            return 0
        lax.fori_loop(0, rows_pt, zdeg, 0)

        eph = NCF * CF

        def chunk(i, flushed):
            eo = pl.multiple_of(c * eph + i * CF, 8)
            pltpu.sync_copy(sh_hbm.at[pl.ds(eo, CF)], sv)
            pltpu.sync_copy(dh_hbm.at[pl.ds(eo, CF)], dv)

            def grp(j, off):
                s16 = sv[pl.ds(j * 16, 16)]
                d16 = dv[pl.ds(j * 16, 16)]
                owner = lax.shift_right_logical(d16, 4) & 15
                m = (owner == s) & (d16 < npad)
                loc = (lax.shift_left(lax.shift_right_logical(d16, 8), 4)
                       | (d16 & 15))
                pc = jnp.cumsum(m.astype(jnp.int32))
                pos = off + pc - 1
                plsc.store_scatter(cbs, (pos,), s16, mask=m)
                plsc.store_scatter(cbl, (pos,), loc, mask=m)
                return off + pc[15]
            cnt = lax.fori_loop(0, CF // 16, grp, 0)

            plsc.store_scatter(cbs, (cnt + iota16,), z16i)
            plsc.store_scatter(cbl, (cnt + iota16,), lp16)

            def dinc(g, _):
                lvec = cbl[pl.ds(g * 16, 16)] * 16
                for kk in range(16):
                    lo = pl.multiple_of(lvec[kk], 16)
                    plsc.addupdate(degv.at[pl.ds(lo, 16)], one16)
                return 0
            lax.fori_loop(0, (cnt + 15) // 16, dinc, 0)

            cnt8 = (cnt + 7) & (-8)
            fo = pl.multiple_of(base + flushed, 8)
            pltpu.sync_copy(cbs, sl_hbm.at[pl.ds(fo, CBN)])
            pltpu.sync_copy(cbl, ll_hbm.at[pl.ds(fo, CBN)])
            return flushed + cnt8
        total = lax.fori_loop(0, NCF, chunk, 0)

        cbs[pl.ds(0, 16)] = z16i
        cbl[pl.ds(0, 16)] = lp16

        def fpad(i, _):
            po = pl.multiple_of(base + total + i * 16, 8)
            pltpu.sync_copy(cbs.at[pl.ds(0, 16)], sl_hbm.at[pl.ds(po, 16)])
            pltpu.sync_copy(cbl.at[pl.ds(0, 16)], ll_hbm.at[pl.ds(po, 16)])
            return 0
        lax.fori_loop(0, 8, fpad, 0)

        cntv[...] = jnp.full((16,), 0, jnp.int32) + total
        pltpu.sync_copy(cntv.at[pl.ds(0, 8)],
                        cnt_hbm.at[pl.ds(pl.multiple_of(w * 8, 8), 8)])
        do = pl.multiple_of(w * rows_pt * 16, 8)
        pltpu.sync_copy(degv, deg_hbm.at[pl.ds(do, rows_pt * 16)])

    return k(sh, dh)


def sc_gather_accum(y, slist, llist, counts, np_acc, d, ch):
    rows_pt = np_acc // NS

    @functools.partial(
        pl.kernel,
        mesh=_sc_mesh(),
        compiler_params=pltpu.CompilerParams(needs_layout_passes=False),
        out_type=jax.ShapeDtypeStruct((NW * rows_pt * d,), jnp.float32),
        scratch_types=[
            pltpu.VMEM((ch,), jnp.int32),
            pltpu.VMEM((ch + 16,), jnp.int32),
            pltpu.VMEM((ch, d), jnp.float32),
            pltpu.VMEM((rows_pt * d,), jnp.float32),
            pltpu.VMEM((16,), jnp.int32),
            pltpu.SemaphoreType.DMA,
        ],
    )
    def k(y_hbm, sl_hbm, ll_hbm, cnt_hbm, out_hbm, sbuf, lbuf, rows, acc,
          cntv, sem):
        c = lax.axis_index("c")
        s = lax.axis_index("s")
        w = c * NS + s
        base = w * CAP

        z16 = jnp.zeros((16,), jnp.float32)

        def zacc(i, _):
            degb = pl.multiple_of(i * d, 8)
            for kk in range(d // 16):
                acc[pl.ds(degb + kk * 16, 16)] = z16
            return 0
        lax.fori_loop(0, rows_pt, zacc, 0)

        pltpu.sync_copy(cnt_hbm.at[pl.ds(pl.multiple_of(w * 8, 8), 8)],
                        cntv.at[pl.ds(0, 8)])
        cnt = cntv[...][0]
        nch = (cnt + (ch - 1)) // ch

        def chunk(i, _):
            co = pl.multiple_of(base + i * ch, 8)
            pltpu.sync_copy(sl_hbm.at[pl.ds(co, ch)], sbuf)
            pltpu.sync_copy(ll_hbm.at[pl.ds(co, ch)],
                            lbuf.at[pl.ds(0, ch)])
            pltpu.async_copy(y_hbm.at[sbuf], rows, sem).wait()

            def grp(g, _):
                lvec = lbuf[pl.ds(g * 16, 16)] * d
                for k in range(16):
                    lb = pl.multiple_of(lvec[k], 8)
                    for kk in range(d // 16):
                        plsc.addupdate(
                            acc.at[pl.ds(lb + kk * 16, 16)],
                            rows[g * 16 + k, pl.ds(kk * 16, 16)])
                return 0
            lax.fori_loop(0, ch // 16, grp, 0)
            return 0
        lax.fori_loop(0, nch, chunk, 0)

        oo = pl.multiple_of(w * rows_pt * d, 8)
        pltpu.sync_copy(acc, out_hbm.at[pl.ds(oo, rows_pt * d)])

    return k(y, slist, llist, counts)


def tc_dinv(deg2, gate):
    npad = deg2.shape[1]

    def body(deg_ref, gate_ref, dinv_ref, s_ref):
        deg = deg_ref[0, :] + deg_ref[1, :] + 1.0
        dinv = lax.rsqrt(deg)
        dinv_ref[0, :] = dinv
        s_ref[0, :] = dinv * gate_ref[0, :]

    return pl.pallas_call(
        body,
        out_shape=(jax.ShapeDtypeStruct((1, npad), jnp.float32),
                   jax.ShapeDtypeStruct((1, npad), jnp.float32)),
    )(deg2, gate)


def tc_matmul_scale(h, w, s3):
    npad, din = h.shape
    dout = w.shape[1]
    nb = npad // BM

    def body(h_ref, w_ref, s_ref, y_ref):
        y = jnp.dot(h_ref[...], w_ref[...], preferred_element_type=jnp.float32)
        y_ref[...] = y * s_ref[0, 0, :][:, None]

    return pl.pallas_call(
        body,
        grid=(nb,),
        in_specs=[
            pl.BlockSpec((BM, din), lambda i: (i, 0)),
            pl.BlockSpec((din, dout), lambda i: (0, 0)),
            pl.BlockSpec((1, 1, BM), lambda i: (i, 0, 0)),
        ],
        out_specs=pl.BlockSpec((BM, dout), lambda i: (i, 0)),
        out_shape=jax.ShapeDtypeStruct((npad, dout), jnp.float32),
    )(h, w, s3)


def tc_epilogue(acc2, y, dinv3, b, p):
    npad, d = y.shape
    nb = npad // BM

    def body(acc_ref, y_ref, dinv_ref, b_ref, p_ref, h_ref, sc_ref):
        a = acc_ref[0] + acc_ref[1] + y_ref[...]
        h = a * dinv_ref[0, 0, :][:, None] + b_ref[0, :][None, :]
        h = jnp.maximum(h, 0.0)
        h_ref[...] = h
        pv = p_ref[0, :]
        pn = pv * lax.rsqrt(jnp.sum(pv * pv))
        sc_ref[0, 0, :] = jnp.tanh(jnp.sum(h * pn[None, :], axis=1))

    return pl.pallas_call(
        body,
        grid=(nb,),
        in_specs=[
            pl.BlockSpec((2, BM, d), lambda i: (0, i, 0)),
            pl.BlockSpec((BM, d), lambda i: (i, 0)),
            pl.BlockSpec((1, 1, BM), lambda i: (i, 0, 0)),
            pl.BlockSpec((1, d), lambda i: (0, 0)),
            pl.BlockSpec((1, d), lambda i: (0, 0)),
        ],
        out_specs=(pl.BlockSpec((BM, d), lambda i: (i, 0)),
                   pl.BlockSpec((1, 1, BM), lambda i: (i, 0, 0))),
        out_shape=(jax.ShapeDtypeStruct((npad, d), jnp.float32),
                   jax.ShapeDtypeStruct((nb, 1, BM), jnp.float32)),
    )(acc2, y, dinv3, b, p)


def tc_readout(h, batch_row, batch_col):
    kpad, d = h.shape

    def body(h_ref, br_ref, bc_ref, z_ref):
        g = pl.program_id(0)
        bmf = (br_ref[...] == g).astype(jnp.float32)
        cnt = jnp.sum(bmf)
        gsum = jnp.dot(bmf, h_ref[...], preferred_element_type=jnp.float32)
        hm = jnp.where(bc_ref[...] == g, h_ref[...], NEG)
        gmx = jnp.max(hm, axis=0, keepdims=True)
        gmx = jnp.where(cnt > 0.0, gmx, 0.0)
        gap = gsum / jnp.maximum(cnt, 1.0)
        z_ref[0] = jnp.concatenate([gmx, gap], axis=1)

    return pl.pallas_call(
        body,
        grid=(NUM_GRAPHS,),
        in_specs=[
            pl.BlockSpec((kpad, d), lambda g: (0, 0)),
            pl.BlockSpec((1, kpad), lambda g: (0, 0)),
            pl.BlockSpec((kpad, 1), lambda g: (0, 0)),
        ],
        out_specs=pl.BlockSpec((1, 1, 2 * d), lambda g: (g, 0, 0)),
        out_shape=jax.ShapeDtypeStruct((NUM_GRAPHS, 1, 2 * d), jnp.float32),
    )(h, batch_row, batch_col)


def tc_mlp(z, wm1, bm1, wm2, bm2):
    g = z.shape[0]
    dc = wm2.shape[1]

    def body(z_ref, w1_ref, b1_ref, w2_ref, b2_ref, o_ref):
        a = jnp.dot(z_ref[...], w1_ref[...],
                    preferred_element_type=jnp.float32) + b1_ref[0, :][None, :]
        a = jnp.maximum(a, 0.0)
        o = jnp.dot(a, w2_ref[...],
                    preferred_element_type=jnp.float32) + b2_ref[0, :][None, :]
        m = jnp.max(o, axis=1, keepdims=True)
        lse = m + jnp.log(jnp.sum(jnp.exp(o - m), axis=1, keepdims=True))
        o_ref[...] = o - lse

    return pl.pallas_call(
        body,
        out_shape=jax.ShapeDtypeStruct((g, dc), jnp.float32),
    )(z, wm1, bm1, wm2, bm2)


def _edges_half(idx, fill):
    pad = jnp.full((EP - idx.shape[0],), fill, jnp.int32)
    return jnp.concatenate([idx.astype(jnp.int32), pad])


def _unravel2(a, np_acc):
    return a.reshape(NC, NS, np_acc // 256, 16).transpose(0, 2, 1, 3).reshape(
        NC, np_acc)


def _unravel3(a, np_acc, d):
    return a.reshape(NC, NS, np_acc // 256, 16, d).transpose(
        0, 2, 1, 3, 4).reshape(NC, np_acc, d)


def gcn_layer(h, gate, sC, dC, w, b, p, n, npad, np_acc):
    dout = w.shape[1]
    sh = _edges_half(sC, 0)
    dh = _edges_half(dC, npad)

    rows_pt = np_acc // NS
    deg_f, slist, llist, counts = sc_partition(sh, dh, np_acc, npad)
    deg_il = deg_f.reshape(NC, NS, rows_pt, 16)[..., 0]
    deg2 = _unravel2(deg_il, np_acc)[:, :npad]
    dinv, s_mm = tc_dinv(deg2, gate.reshape(1, npad))
    nb = npad // BM
    y = tc_matmul_scale(h, w, s_mm.reshape(nb, 1, BM))
    acc_f = sc_gather_accum(y, slist, llist, counts, np_acc, dout,
                            _chunk_for(dout))
    acc2 = _unravel3(acc_f.reshape(NC, NS, rows_pt, dout), np_acc, dout)
    hn, sc3 = tc_epilogue(acc2, y, dinv.reshape(nb, 1, BM),
                          b.reshape(1, -1), p.reshape(1, -1))
    score = sc3.reshape(npad)
    score = jnp.where(jnp.arange(npad) < n, score, NEG)
    return hn, score


def kernel(x, edge_index, batch, epoch, W1, b1, p1, W2, b2, p2, W3, b3,
           Wm1, bm1, Wm2, bm2):
    del epoch
    n1p, k1p, k2p = _pad_rows(N0), _pad_rows(K1), _pad_rows(K2)
    a1, a2, a3 = _pad_acc(N0), _pad_acc(K1), _pad_acc(K2)

    src = edge_index[0].astype(jnp.int32)
    dst = edge_index[1].astype(jnp.int32)
    batch = batch.astype(jnp.int32)

    xp = jnp.pad(x, ((0, n1p - N0), (0, 0)))
    g1 = jnp.where(jnp.arange(n1p) < N0, 1.0, 0.0).astype(jnp.float32)

    h1, sc1 = gcn_layer(xp, g1, src, dst, W1, b1, p1, N0, n1p, a1)

    vals1, perm1 = lax.top_k(sc1, K1)
    h1p = jnp.pad(h1[perm1], ((0, k1p - K1), (0, 0)))
    gate2 = jnp.pad(vals1, (0, k1p - K1))
    batch2 = jnp.pad(batch[perm1], (0, k1p - K1), constant_values=NUM_GRAPHS)
    nidx1 = jnp.full((n1p,), -1, jnp.int32).at[perm1].set(
        jnp.arange(K1, dtype=jnp.int32))
    s2 = nidx1[src]
    d2 = nidx1[dst]
    m2 = (s2 >= 0) & (d2 >= 0)
    s2c = jnp.where(m2, s2, 0)
    d2c = jnp.where(m2, d2, k1p)

    h2, sc2 = gcn_layer(h1p, gate2, s2c, d2c, W2, b2, p2, K1, k1p, a2)

    vals2, perm2 = lax.top_k(sc2, K2)
    h2p = jnp.pad(h2[perm2], ((0, k2p - K2), (0, 0)))
    gate3 = jnp.pad(vals2, (0, k2p - K2))
    batch3 = jnp.pad(batch2[perm2], (0, k2p - K2), constant_values=NUM_GRAPHS)
    nidx2 = jnp.full((k1p,), -1, jnp.int32).at[perm2].set(
        jnp.arange(K2, dtype=jnp.int32))
    s3 = nidx2[s2]
    d3 = nidx2[d2]
    m3 = m2 & (s3 >= 0) & (d3 >= 0)
    s3c = jnp.where(m3, s3, 0)
    d3c = jnp.where(m3, d3, k2p)

    h3, _ = gcn_layer(h2p, gate3, s3c, d3c, W3, b3,
                      jnp.ones((W3.shape[1],), jnp.float32), K2, k2p, a3)

    z = tc_readout(h3, batch3.reshape(1, k2p),
                   batch3.reshape(k2p, 1)).reshape(NUM_GRAPHS, -1)
    return tc_mlp(z, Wm1, bm1.reshape(1, -1), Wm2, bm2.reshape(1, -1))


# speedup vs baseline: 14.9470x; 3.8620x over previous
import functools
import math

import jax
import jax.numpy as jnp
from jax import lax
from jax.experimental import pallas as pl
from jax.experimental.pallas import tpu as pltpu
from jax.experimental.pallas import tpu_sc as plsc

N0 = 10000
NUM_GRAPHS = 64
E0 = 320000
K1 = int(math.ceil(0.5 * N0))
K2 = int(math.ceil(0.5 * K1))

NC = 2
NS = 16
NW = NC * NS
EP = 327680
BM = 256

NEG = -3.0e38


def _pad_rows(n):
    return ((n + BM - 1) // BM) * BM


def _pad_acc(n):
    return _pad_rows(n) + 256


def _chunk_for(d):
    return {128: 128, 256: 128, 512: 64}[d]


def _sc_mesh():
    return plsc.VectorSubcoreMesh(core_axis_name="c", subcore_axis_name="s",
                                  num_cores=NC, num_subcores=NS)


CF = 2048
NCF = EP // (NC * CF)
CAP = 167936
CBN = CF + 128


def sc_partition(sh, dh, np_acc, npad):
    rows_pt = np_acc // NS
    locpad = rows_pt - 16

    @functools.partial(
        pl.kernel,
        mesh=_sc_mesh(),
        compiler_params=pltpu.CompilerParams(needs_layout_passes=False),
        out_type=(jax.ShapeDtypeStruct((NW * rows_pt * 16,), jnp.float32),
                  jax.ShapeDtypeStruct((NW * CAP,), jnp.int32),
                  jax.ShapeDtypeStruct((NW * CAP,), jnp.int32),
                  jax.ShapeDtypeStruct((NW * 8,), jnp.int32)),
        scratch_types=[
            pltpu.VMEM((CF,), jnp.int32),
            pltpu.VMEM((CF,), jnp.int32),
            pltpu.VMEM((CBN,), jnp.int32),
            pltpu.VMEM((CBN,), jnp.int32),
            pltpu.VMEM((rows_pt * 16,), jnp.float32),
            pltpu.VMEM((16,), jnp.int32),
        ],
    )
    def k(sh_hbm, dh_hbm, deg_hbm, sl_hbm, ll_hbm, cnt_hbm,
          sv, dv, cbs, cbl, degv, cntv):
        c = lax.axis_index("c")
        s = lax.axis_index("s")
        w = c * NS + s
        base = w * CAP

        z16 = jnp.zeros((16,), jnp.float32)
        one16 = (lax.iota(jnp.int32, 16) == 0).astype(jnp.float32)
        iota16 = lax.iota(jnp.int32, 16)
        lp16 = jnp.full((16,), locpad, jnp.int32)
        z16i = jnp.zeros((16,), jnp.int32)

        def zdeg(i, _):
            degv[pl.ds(i * 16, 16)] = z16
            return 0
        lax.fori_loop(0, rows_pt, zdeg, 0)

        eph = NCF * CF

        def chunk(i, flushed):
            eo = pl.multiple_of(c * eph + i * CF, 8)
            pltpu.sync_copy(sh_hbm.at[pl.ds(eo, CF)], sv)
            pltpu.sync_copy(dh_hbm.at[pl.ds(eo, CF)], dv)

            def grp(j, off):
                s16 = sv[pl.ds(j * 16, 16)]
                d16 = dv[pl.ds(j * 16, 16)]
                owner = lax.shift_right_logical(d16, 4) & 15
                m = (owner == s) & (d16 < npad)
                loc = (lax.shift_left(lax.shift_right_logical(d16, 8), 4)
                       | (d16 & 15))
                pc = jnp.cumsum(m.astype(jnp.int32))
                pos = off + pc - 1
                plsc.store_scatter(cbs, (pos,), s16, mask=m)
                plsc.store_scatter(cbl, (pos,), loc, mask=m)
                return off + pc[15]
            cnt = lax.fori_loop(0, CF // 16, grp, 0)

            plsc.store_scatter(cbs, (cnt + iota16,), z16i)
            plsc.store_scatter(cbl, (cnt + iota16,), lp16)

            def dinc(g, _):
                lvec = cbl[pl.ds(g * 16, 16)] * 16
                for kk in range(16):
                    lo = pl.multiple_of(lvec[kk], 16)
                    plsc.addupdate(degv.at[pl.ds(lo, 16)], one16)
                return 0
            lax.fori_loop(0, (cnt + 15) // 16, dinc, 0)

            cnt8 = (cnt + 7) & (-8)
            fo = pl.multiple_of(base + flushed, 8)
            pltpu.sync_copy(cbs, sl_hbm.at[pl.ds(fo, CBN)])
            pltpu.sync_copy(cbl, ll_hbm.at[pl.ds(fo, CBN)])
            return flushed + cnt8
        total = lax.fori_loop(0, NCF, chunk, 0)

        cbs[pl.ds(0, 16)] = z16i
        cbl[pl.ds(0, 16)] = lp16

        def fpad(i, _):
            po = pl.multiple_of(base + total + i * 16, 8)
            pltpu.sync_copy(cbs.at[pl.ds(0, 16)], sl_hbm.at[pl.ds(po, 16)])
            pltpu.sync_copy(cbl.at[pl.ds(0, 16)], ll_hbm.at[pl.ds(po, 16)])
            return 0
        lax.fori_loop(0, 8, fpad, 0)

        cntv[...] = jnp.full((16,), 0, jnp.int32) + total
        pltpu.sync_copy(cntv.at[pl.ds(0, 8)],
                        cnt_hbm.at[pl.ds(pl.multiple_of(w * 8, 8), 8)])
        do = pl.multiple_of(w * rows_pt * 16, 8)
        pltpu.sync_copy(degv, deg_hbm.at[pl.ds(do, rows_pt * 16)])

    return k(sh, dh)


def sc_gather_accum(y, slist, llist, counts, np_acc, d, ch):
    rows_pt = np_acc // NS

    @functools.partial(
        pl.kernel,
        mesh=_sc_mesh(),
        compiler_params=pltpu.CompilerParams(needs_layout_passes=False),
        out_type=jax.ShapeDtypeStruct((NW * rows_pt * d,), jnp.float32),
        scratch_types=[
            pltpu.VMEM((ch,), jnp.int32),
            pltpu.VMEM((ch + 16,), jnp.int32),
            pltpu.VMEM((ch, d), jnp.float32),
            pltpu.VMEM((rows_pt * d,), jnp.float32),
            pltpu.VMEM((16,), jnp.int32),
            pltpu.SemaphoreType.DMA,
        ],
    )
    def k(y_hbm, sl_hbm, ll_hbm, cnt_hbm, out_hbm, sbuf, lbuf, rows, acc,
          cntv, sem):
        c = lax.axis_index("c")
        s = lax.axis_index("s")
        w = c * NS + s
        base = w * CAP

        z16 = jnp.zeros((16,), jnp.float32)

        def zacc(i, _):
            degb = pl.multiple_of(i * d, 8)
            for kk in range(d // 16):
                acc[pl.ds(degb + kk * 16, 16)] = z16
            return 0
        lax.fori_loop(0, rows_pt, zacc, 0)

        pltpu.sync_copy(cnt_hbm.at[pl.ds(pl.multiple_of(w * 8, 8), 8)],
                        cntv.at[pl.ds(0, 8)])
        cnt = cntv[...][0]
        nch = (cnt + (ch - 1)) // ch

        def chunk(i, _):
            co = pl.multiple_of(base + i * ch, 8)
            pltpu.sync_copy(sl_hbm.at[pl.ds(co, ch)], sbuf)
            pltpu.sync_copy(ll_hbm.at[pl.ds(co, ch)],
                            lbuf.at[pl.ds(0, ch)])
            pltpu.async_copy(y_hbm.at[sbuf], rows, sem).wait()

            def grp(g, _):
                lvec = lbuf[pl.ds(g * 16, 16)] * d
                for k in range(16):
                    lb = pl.multiple_of(lvec[k], 8)
                    for kk in range(d // 16):
                        plsc.addupdate(
                            acc.at[pl.ds(lb + kk * 16, 16)],
                            rows[g * 16 + k, pl.ds(kk * 16, 16)])
                return 0
            lax.fori_loop(0, ch // 16, grp, 0)
            return 0
        lax.fori_loop(0, nch, chunk, 0)

        oo = pl.multiple_of(w * rows_pt * d, 8)
        pltpu.sync_copy(acc, out_hbm.at[pl.ds(oo, rows_pt * d)])

    return k(y, slist, llist, counts)


CR = 2048
NER = EP // NW
NCR = NER // CR


def sc_remap(perm, btab, s_ep, d_ep, npad_prev, npad_next, kpad):
    kb = kpad // NW

    @functools.partial(
        pl.kernel,
        mesh=_sc_mesh(),
        compiler_params=pltpu.CompilerParams(needs_layout_passes=False),
        out_type=(jax.ShapeDtypeStruct((EP,), jnp.int32),
                  jax.ShapeDtypeStruct((EP,), jnp.int32),
                  jax.ShapeDtypeStruct((kpad,), jnp.int32)),
        scratch_types=[
            pltpu.VMEM((kpad,), jnp.int32),
            pltpu.VMEM((npad_prev,), jnp.int32),
            pltpu.VMEM((npad_prev,), jnp.int32),
            pltpu.VMEM((CR,), jnp.int32),
            pltpu.VMEM((CR,), jnp.int32),
            pltpu.VMEM((CR,), jnp.int32),
            pltpu.VMEM((CR,), jnp.int32),
            pltpu.VMEM((kb,), jnp.int32),
        ],
    )
    def k(perm_hbm, btab_hbm, s_hbm, d_hbm, sn_hbm, dn_hbm, bn_hbm,
          pb, nidx, bt, sv, dv, so, do, bo):
        c = lax.axis_index("c")
        s = lax.axis_index("s")
        w = c * NS + s

        iota16 = lax.iota(jnp.int32, 16)
        neg16 = jnp.full((16,), -1, jnp.int32)

        pltpu.sync_copy(perm_hbm, pb)
        pltpu.sync_copy(btab_hbm, bt)

        def zn(i, _):
            nidx[pl.ds(i * 16, 16)] = neg16
            return 0
        lax.fori_loop(0, npad_prev // 16, zn, 0)

        def sca(g, _):
            pv = pb[pl.ds(g * 16, 16)]
            plsc.store_scatter(nidx, (pv,), g * 16 + iota16)
            return 0
        lax.fori_loop(0, kpad // 16, sca, 0)

        def chunk(i, _):
            eo = pl.multiple_of(w * NER + i * CR, 8)
            pltpu.sync_copy(s_hbm.at[pl.ds(eo, CR)], sv)
            pltpu.sync_copy(d_hbm.at[pl.ds(eo, CR)], dv)

            def grp(j, _):
                s16 = sv[pl.ds(j * 16, 16)]
                d16 = dv[pl.ds(j * 16, 16)]
                valid = d16 < npad_prev
                dcl = jnp.where(valid, d16, 0)
                sg = plsc.load_gather(nidx, (s16,))
                dg = plsc.load_gather(nidx, (dcl,))
                m = valid & (sg >= 0) & (dg >= 0)
                so[pl.ds(j * 16, 16)] = jnp.where(m, sg, 0)
                do[pl.ds(j * 16, 16)] = jnp.where(m, dg, npad_next)
                return 0
            lax.fori_loop(0, CR // 16, grp, 0)

            pltpu.sync_copy(so, sn_hbm.at[pl.ds(eo, CR)])
            pltpu.sync_copy(do, dn_hbm.at[pl.ds(eo, CR)])
            return 0
        lax.fori_loop(0, NCR, chunk, 0)

        bb = w * kb

        def bg(g, _):
            pv = pb[pl.ds(bb + g * 16, 16)]
            bo[pl.ds(g * 16, 16)] = plsc.load_gather(bt, (pv,))
            return 0
        lax.fori_loop(0, kb // 16, bg, 0)
        pltpu.sync_copy(bo, bn_hbm.at[pl.ds(pl.multiple_of(bb, 8), kb)])

    return k(perm, btab, s_ep, d_ep)


def tc_dinv(deg2, gate):
    npad = deg2.shape[1]

    def body(deg_ref, gate_ref, dinv_ref, s_ref):
        deg = deg_ref[0, :] + deg_ref[1, :] + 1.0
        dinv = lax.rsqrt(deg)
        dinv_ref[0, :] = dinv
        s_ref[0, :] = dinv * gate_ref[0, :]

    return pl.pallas_call(
        body,
        out_shape=(jax.ShapeDtypeStruct((1, npad), jnp.float32),
                   jax.ShapeDtypeStruct((1, npad), jnp.float32)),
    )(deg2, gate)


def tc_matmul_scale(h, w, s3):
    npad, din = h.shape
    dout = w.shape[1]
    nb = npad // BM

    def body(h_ref, w_ref, s_ref, y_ref):
        y = jnp.dot(h_ref[...], w_ref[...], preferred_element_type=jnp.float32)
        y_ref[...] = y * s_ref[0, 0, :][:, None]

    return pl.pallas_call(
        body,
        grid=(nb,),
        in_specs=[
            pl.BlockSpec((BM, din), lambda i: (i, 0)),
            pl.BlockSpec((din, dout), lambda i: (0, 0)),
            pl.BlockSpec((1, 1, BM), lambda i: (i, 0, 0)),
        ],
        out_specs=pl.BlockSpec((BM, dout), lambda i: (i, 0)),
        out_shape=jax.ShapeDtypeStruct((npad, dout), jnp.float32),
    )(h, w, s3)


def tc_epilogue(acc2, y, dinv3, b, p):
    npad, d = y.shape
    nb = npad // BM

    def body(acc_ref, y_ref, dinv_ref, b_ref, p_ref, h_ref, sc_ref):
        a = acc_ref[0] + acc_ref[1] + y_ref[...]
        h = a * dinv_ref[0, 0, :][:, None] + b_ref[0, :][None, :]
        h = jnp.maximum(h, 0.0)
        h_ref[...] = h
        pv = p_ref[0, :]
        pn = pv * lax.rsqrt(jnp.sum(pv * pv))
        sc_ref[0, 0, :] = jnp.tanh(jnp.sum(h * pn[None, :], axis=1))

    return pl.pallas_call(
        body,
        grid=(nb,),
        in_specs=[
            pl.BlockSpec((2, BM, d), lambda i: (0, i, 0)),
            pl.BlockSpec((BM, d), lambda i: (i, 0)),
            pl.BlockSpec((1, 1, BM), lambda i: (i, 0, 0)),
            pl.BlockSpec((1, d), lambda i: (0, 0)),
            pl.BlockSpec((1, d), lambda i: (0, 0)),
        ],
        out_specs=(pl.BlockSpec((BM, d), lambda i: (i, 0)),
                   pl.BlockSpec((1, 1, BM), lambda i: (i, 0, 0))),
        out_shape=(jax.ShapeDtypeStruct((npad, d), jnp.float32),
                   jax.ShapeDtypeStruct((nb, 1, BM), jnp.float32)),
    )(acc2, y, dinv3, b, p)


def tc_readout(h, batch_row, batch_col):
    kpad, d = h.shape

    def body(h_ref, br_ref, bc_ref, z_ref):
        g = pl.program_id(0)
        bmf = (br_ref[...] == g).astype(jnp.float32)
        cnt = jnp.sum(bmf)
        gsum = jnp.dot(bmf, h_ref[...], preferred_element_type=jnp.float32)
        hm = jnp.where(bc_ref[...] == g, h_ref[...], NEG)
        gmx = jnp.max(hm, axis=0, keepdims=True)
        gmx = jnp.where(cnt > 0.0, gmx, 0.0)
        gap = gsum / jnp.maximum(cnt, 1.0)
        z_ref[0] = jnp.concatenate([gmx, gap], axis=1)

    return pl.pallas_call(
        body,
        grid=(NUM_GRAPHS,),
        in_specs=[
            pl.BlockSpec((kpad, d), lambda g: (0, 0)),
            pl.BlockSpec((1, kpad), lambda g: (0, 0)),
            pl.BlockSpec((kpad, 1), lambda g: (0, 0)),
        ],
        out_specs=pl.BlockSpec((1, 1, 2 * d), lambda g: (g, 0, 0)),
        out_shape=jax.ShapeDtypeStruct((NUM_GRAPHS, 1, 2 * d), jnp.float32),
    )(h, batch_row, batch_col)


def tc_mlp(z, wm1, bm1, wm2, bm2):
    g = z.shape[0]
    dc = wm2.shape[1]

    def body(z_ref, w1_ref, b1_ref, w2_ref, b2_ref, o_ref):
        a = jnp.dot(z_ref[...], w1_ref[...],
                    preferred_element_type=jnp.float32) + b1_ref[0, :][None, :]
        a = jnp.maximum(a, 0.0)
        o = jnp.dot(a, w2_ref[...],
                    preferred_element_type=jnp.float32) + b2_ref[0, :][None, :]
        m = jnp.max(o, axis=1, keepdims=True)
        lse = m + jnp.log(jnp.sum(jnp.exp(o - m), axis=1, keepdims=True))
        o_ref[...] = o - lse

    return pl.pallas_call(
        body,
        out_shape=jax.ShapeDtypeStruct((g, dc), jnp.float32),
    )(z, wm1, bm1, wm2, bm2)


def _edges_half(idx, fill):
    pad = jnp.full((EP - idx.shape[0],), fill, jnp.int32)
    return jnp.concatenate([idx.astype(jnp.int32), pad])


def _unravel2(a, np_acc):
    return a.reshape(NC, NS, np_acc // 256, 16).transpose(0, 2, 1, 3).reshape(
        NC, np_acc)


def _unravel3(a, np_acc, d):
    return a.reshape(NC, NS, np_acc // 256, 16, d).transpose(
        0, 2, 1, 3, 4).reshape(NC, np_acc, d)


def gcn_layer(h, gate, sh, dh, w, b, p, n, npad, np_acc):
    dout = w.shape[1]
    rows_pt = np_acc // NS
    deg_f, slist, llist, counts = sc_partition(sh, dh, np_acc, npad)
    deg_il = deg_f.reshape(NC, NS, rows_pt, 16)[..., 0]
    deg2 = _unravel2(deg_il, np_acc)[:, :npad]
    dinv, s_mm = tc_dinv(deg2, gate.reshape(1, npad))
    nb = npad // BM
    y = tc_matmul_scale(h, w, s_mm.reshape(nb, 1, BM))
    acc_f = sc_gather_accum(y, slist, llist, counts, np_acc, dout,
                            _chunk_for(dout))
    acc2 = _unravel3(acc_f.reshape(NC, NS, rows_pt, dout), np_acc, dout)
    hn, sc3 = tc_epilogue(acc2, y, dinv.reshape(nb, 1, BM),
                          b.reshape(1, -1), p.reshape(1, -1))
    score = sc3.reshape(npad)
    score = jnp.where(jnp.arange(npad) < n, score, NEG)
    return hn, score


def kernel(x, edge_index, batch, epoch, W1, b1, p1, W2, b2, p2, W3, b3,
           Wm1, bm1, Wm2, bm2):
    del epoch
    n1p, k1p, k2p = _pad_rows(N0), _pad_rows(K1), _pad_rows(K2)
    a1, a2, a3 = _pad_acc(N0), _pad_acc(K1), _pad_acc(K2)

    src = edge_index[0].astype(jnp.int32)
    dst = edge_index[1].astype(jnp.int32)
    batch = batch.astype(jnp.int32)

    xp = jnp.pad(x, ((0, n1p - N0), (0, 0)))
    g1 = jnp.where(jnp.arange(n1p) < N0, 1.0, 0.0).astype(jnp.float32)
    batch1 = jnp.pad(batch, (0, n1p - N0), constant_values=NUM_GRAPHS)
    s1 = _edges_half(src, 0)
    d1 = _edges_half(dst, n1p)

    h1, sc1 = gcn_layer(xp, g1, s1, d1, W1, b1, p1, N0, n1p, a1)

    vals1, perm1 = lax.top_k(sc1, K1)
    permp1 = jnp.pad(perm1, (0, k1p - K1), constant_values=N0)
    gate2 = jnp.pad(vals1, (0, k1p - K1))
    s2, d2, batch2 = sc_remap(permp1, batch1, s1, d1, n1p, k1p, k1p)
    h1p = h1[permp1]

    h2, sc2 = gcn_layer(h1p, gate2, s2, d2, W2, b2, p2, K1, k1p, a2)

    vals2, perm2 = lax.top_k(sc2, K2)
    permp2 = jnp.pad(perm2, (0, k2p - K2), constant_values=K1)
    gate3 = jnp.pad(vals2, (0, k2p - K2))
    s3, d3, batch3 = sc_remap(permp2, batch2, s2, d2, k1p, k2p, k2p)
    h2p = h2[permp2]

    h3, _ = gcn_layer(h2p, gate3, s3, d3, W3, b3,
                      jnp.ones((W3.shape[1],), jnp.float32), K2, k2p, a3)

    z = tc_readout(h3, batch3.reshape(1, k2p),
                   batch3.reshape(k2p, 1)).reshape(NUM_GRAPHS, -1)
    return tc_mlp(z, Wm1, bm1.reshape(1, -1), Wm2, bm2.reshape(1, -1))


# speedup vs baseline: 19.0079x; 1.2717x over previous
import functools
import math

import jax
import jax.numpy as jnp
from jax import lax
from jax.experimental import pallas as pl
from jax.experimental.pallas import tpu as pltpu
from jax.experimental.pallas import tpu_sc as plsc

N0 = 10000
NUM_GRAPHS = 64
E0 = 320000
K1 = int(math.ceil(0.5 * N0))
K2 = int(math.ceil(0.5 * K1))

NC = 2
NS = 16
NW = NC * NS
EP = 327680
BM = 256

NEG = -3.0e38


def _pad_rows(n):
    return ((n + BM - 1) // BM) * BM


def _pad_acc(n):
    return _pad_rows(n) + 256


def _chunk_for(d):
    return {128: 128, 256: 128, 512: 64}[d]


def _sc_mesh():
    return plsc.VectorSubcoreMesh(core_axis_name="c", subcore_axis_name="s",
                                  num_cores=NC, num_subcores=NS)


CF = 8192
NCF = EP // (NC * CF)
CAP = 167936
CBH = CF // 2 + 64
CBN = 2 * CBH


def sc_partition(sh, dh, np_acc, npad):
    rows_pt = np_acc // NS
    locpad = rows_pt - 16

    @functools.partial(
        pl.kernel,
        mesh=_sc_mesh(),
        compiler_params=pltpu.CompilerParams(needs_layout_passes=False),
        out_type=(jax.ShapeDtypeStruct((NW * rows_pt * 16,), jnp.float32),
                  jax.ShapeDtypeStruct((NW * CAP,), jnp.int32),
                  jax.ShapeDtypeStruct((NW * CAP,), jnp.int32),
                  jax.ShapeDtypeStruct((NW * 8,), jnp.int32)),
        scratch_types=[
            pltpu.VMEM((CF,), jnp.int32),
            pltpu.VMEM((CF,), jnp.int32),
            pltpu.VMEM((CBN,), jnp.int32),
            pltpu.VMEM((CBN,), jnp.int32),
            pltpu.VMEM((rows_pt * 16,), jnp.float32),
            pltpu.VMEM((16,), jnp.int32),
        ],
    )
    def k(sh_hbm, dh_hbm, deg_hbm, sl_hbm, ll_hbm, cnt_hbm,
          sv, dv, cbs, cbl, degv, cntv):
        c = lax.axis_index("c")
        s = lax.axis_index("s")
        w = c * NS + s
        base = w * CAP

        z16 = jnp.zeros((16,), jnp.float32)
        one16 = (lax.iota(jnp.int32, 16) == 0).astype(jnp.float32)
        iota16 = lax.iota(jnp.int32, 16)
        lp16 = jnp.full((16,), locpad, jnp.int32)
        z16i = jnp.zeros((16,), jnp.int32)

        def zdeg(i, _):
            degv[pl.ds(i * 16, 16)] = z16
            return 0
        lax.fori_loop(0, rows_pt, zdeg, 0)

        eph = NCF * CF

        ng2 = CF // 32

        def chunk(i, flushed):
            eo = pl.multiple_of(c * eph + i * CF, 8)
            pltpu.sync_copy(sh_hbm.at[pl.ds(eo, CF)], sv)
            pltpu.sync_copy(dh_hbm.at[pl.ds(eo, CF)], dv)

            def filt16(j, off, obase):
                s16 = sv[pl.ds(j * 16, 16)]
                d16 = dv[pl.ds(j * 16, 16)]
                owner = lax.shift_right_logical(d16, 4) & 15
                m = (owner == s) & (d16 < npad)
                loc = (lax.shift_left(lax.shift_right_logical(d16, 8), 4)
                       | (d16 & 15))
                pc = jnp.cumsum(m.astype(jnp.int32))
                pos = obase + off + pc - 1
                plsc.store_scatter(cbs, (pos,), s16, mask=m)
                plsc.store_scatter(cbl, (pos,), loc, mask=m)
                return off + pc[15]

            def grp(j, offs):
                oa, ob = offs
                oa = filt16(j, oa, 0)
                ob = filt16(ng2 + j, ob, CBH)
                return (oa, ob)
            cnt_a, cnt_b = lax.fori_loop(0, ng2, grp, (0, 0))

            plsc.store_scatter(cbs, (cnt_a + iota16,), z16i)
            plsc.store_scatter(cbl, (cnt_a + iota16,), lp16)
            plsc.store_scatter(cbs, (CBH + cnt_b + iota16,), z16i)
            plsc.store_scatter(cbl, (CBH + cnt_b + iota16,), lp16)

            na = (cnt_a + 15) // 16
            nb = (cnt_b + 15) // 16
            lppad = locpad * 16

            def dinc(g, _):
                lvec = jnp.where(g < na, cbl[pl.ds(g * 16, 16)] * 16, lppad)
                lvec2 = jnp.where(g < nb,
                                  cbl[pl.ds(CBH + g * 16, 16)] * 16, lppad)
                for kk in range(16):
                    lo = pl.multiple_of(lvec[kk], 16)
                    plsc.addupdate(degv.at[pl.ds(lo, 16)], one16)
                    lo2 = pl.multiple_of(lvec2[kk], 16)
                    plsc.addupdate(degv.at[pl.ds(lo2, 16)], one16)
                return 0
            lax.fori_loop(0, jnp.maximum(na, nb), dinc, 0)

            ca8 = (cnt_a + 7) & (-8)
            cb8 = (cnt_b + 7) & (-8)
            fo = pl.multiple_of(base + flushed, 8)
            pltpu.sync_copy(cbs.at[pl.ds(0, CBH)], sl_hbm.at[pl.ds(fo, CBH)])
            pltpu.sync_copy(cbl.at[pl.ds(0, CBH)], ll_hbm.at[pl.ds(fo, CBH)])
            fo2 = pl.multiple_of(base + flushed + ca8, 8)
            pltpu.sync_copy(cbs.at[pl.ds(CBH, CBH)],
                            sl_hbm.at[pl.ds(fo2, CBH)])
            pltpu.sync_copy(cbl.at[pl.ds(CBH, CBH)],
                            ll_hbm.at[pl.ds(fo2, CBH)])
            return flushed + ca8 + cb8
        total = lax.fori_loop(0, NCF, chunk, 0)

        cbs[pl.ds(0, 16)] = z16i
        cbl[pl.ds(0, 16)] = lp16

        def fpad(i, _):
            po = pl.multiple_of(base + total + i * 16, 8)
            pltpu.sync_copy(cbs.at[pl.ds(0, 16)], sl_hbm.at[pl.ds(po, 16)])
            pltpu.sync_copy(cbl.at[pl.ds(0, 16)], ll_hbm.at[pl.ds(po, 16)])
            return 0
        lax.fori_loop(0, 8, fpad, 0)

        cntv[...] = jnp.full((16,), 0, jnp.int32) + total
        pltpu.sync_copy(cntv.at[pl.ds(0, 8)],
                        cnt_hbm.at[pl.ds(pl.multiple_of(w * 8, 8), 8)])
        do = pl.multiple_of(w * rows_pt * 16, 8)
        pltpu.sync_copy(degv, deg_hbm.at[pl.ds(do, rows_pt * 16)])

    return k(sh, dh)


def sc_gather_accum(y, slist, llist, counts, np_acc, d, ch):
    rows_pt = np_acc // NS

    @functools.partial(
        pl.kernel,
        mesh=_sc_mesh(),
        compiler_params=pltpu.CompilerParams(needs_layout_passes=False),
        out_type=jax.ShapeDtypeStruct((NW * rows_pt * d,), jnp.float32),
        scratch_types=[
            pltpu.VMEM((ch,), jnp.int32),
            pltpu.VMEM((ch + 16,), jnp.int32),
            pltpu.VMEM((ch, d), jnp.float32),
            pltpu.VMEM((rows_pt * d,), jnp.float32),
            pltpu.VMEM((16,), jnp.int32),
            pltpu.SemaphoreType.DMA,
        ],
    )
    def k(y_hbm, sl_hbm, ll_hbm, cnt_hbm, out_hbm, sbuf, lbuf, rows, acc,
          cntv, sem):
        c = lax.axis_index("c")
        s = lax.axis_index("s")
        w = c * NS + s
        base = w * CAP

        z16 = jnp.zeros((16,), jnp.float32)

        def zacc(i, _):
            degb = pl.multiple_of(i * d, 8)
            for kk in range(d // 16):
                acc[pl.ds(degb + kk * 16, 16)] = z16
            return 0
        lax.fori_loop(0, rows_pt, zacc, 0)

        pltpu.sync_copy(cnt_hbm.at[pl.ds(pl.multiple_of(w * 8, 8), 8)],
                        cntv.at[pl.ds(0, 8)])
        cnt = cntv[...][0]
        nch = (cnt + (ch - 1)) // ch

        def chunk(i, _):
            co = pl.multiple_of(base + i * ch, 8)
            pltpu.sync_copy(sl_hbm.at[pl.ds(co, ch)], sbuf)
            pltpu.sync_copy(ll_hbm.at[pl.ds(co, ch)],
                            lbuf.at[pl.ds(0, ch)])
            pltpu.async_copy(y_hbm.at[sbuf], rows, sem).wait()

            def grp(g, _):
                lvec = lbuf[pl.ds(g * 16, 16)] * d
                for k in range(16):
                    lb = pl.multiple_of(lvec[k], 8)
                    for kk in range(d // 16):
                        plsc.addupdate(
                            acc.at[pl.ds(lb + kk * 16, 16)],
                            rows[g * 16 + k, pl.ds(kk * 16, 16)])
                return 0
            lax.fori_loop(0, ch // 16, grp, 0)
            return 0
        lax.fori_loop(0, nch, chunk, 0)

        oo = pl.multiple_of(w * rows_pt * d, 8)
        pltpu.sync_copy(acc, out_hbm.at[pl.ds(oo, rows_pt * d)])

    return k(y, slist, llist, counts)


CR = 2048
NER = EP // NW
NCR = NER // CR


def sc_remap(perm, btab, s_ep, d_ep, npad_prev, npad_next, kpad):
    kb = kpad // NW

    @functools.partial(
        pl.kernel,
        mesh=_sc_mesh(),
        compiler_params=pltpu.CompilerParams(needs_layout_passes=False),
        out_type=(jax.ShapeDtypeStruct((EP,), jnp.int32),
                  jax.ShapeDtypeStruct((EP,), jnp.int32),
                  jax.ShapeDtypeStruct((kpad,), jnp.int32)),
        scratch_types=[
            pltpu.VMEM((kpad,), jnp.int32),
            pltpu.VMEM((npad_prev,), jnp.int32),
            pltpu.VMEM((npad_prev,), jnp.int32),
            pltpu.VMEM((CR,), jnp.int32),
            pltpu.VMEM((CR,), jnp.int32),
            pltpu.VMEM((CR,), jnp.int32),
            pltpu.VMEM((CR,), jnp.int32),
            pltpu.VMEM((kb,), jnp.int32),
        ],
    )
    def k(perm_hbm, btab_hbm, s_hbm, d_hbm, sn_hbm, dn_hbm, bn_hbm,
          pb, nidx, bt, sv, dv, so, do, bo):
        c = lax.axis_index("c")
        s = lax.axis_index("s")
        w = c * NS + s

        iota16 = lax.iota(jnp.int32, 16)
        neg16 = jnp.full((16,), -1, jnp.int32)

        pltpu.sync_copy(perm_hbm, pb)
        pltpu.sync_copy(btab_hbm, bt)

        def zn(i, _):
            nidx[pl.ds(i * 16, 16)] = neg16
            return 0
        lax.fori_loop(0, npad_prev // 16, zn, 0)

        def sca(g, _):
            pv = pb[pl.ds(g * 16, 16)]
            plsc.store_scatter(nidx, (pv,), g * 16 + iota16)
            return 0
        lax.fori_loop(0, kpad // 16, sca, 0)

        def chunk(i, _):
            eo = pl.multiple_of(w * NER + i * CR, 8)
            pltpu.sync_copy(s_hbm.at[pl.ds(eo, CR)], sv)
            pltpu.sync_copy(d_hbm.at[pl.ds(eo, CR)], dv)

            def grp(j, _):
                s16 = sv[pl.ds(j * 16, 16)]
                d16 = dv[pl.ds(j * 16, 16)]
                valid = d16 < npad_prev
                dcl = jnp.where(valid, d16, 0)
                sg = plsc.load_gather(nidx, (s16,))
                dg = plsc.load_gather(nidx, (dcl,))
                m = valid & (sg >= 0) & (dg >= 0)
                so[pl.ds(j * 16, 16)] = jnp.where(m, sg, 0)
                do[pl.ds(j * 16, 16)] = jnp.where(m, dg, npad_next)
                return 0
            lax.fori_loop(0, CR // 16, grp, 0)

            pltpu.sync_copy(so, sn_hbm.at[pl.ds(eo, CR)])
            pltpu.sync_copy(do, dn_hbm.at[pl.ds(eo, CR)])
            return 0
        lax.fori_loop(0, NCR, chunk, 0)

        bb = w * kb

        def bg(g, _):
            pv = pb[pl.ds(bb + g * 16, 16)]
            bo[pl.ds(g * 16, 16)] = plsc.load_gather(bt, (pv,))
            return 0
        lax.fori_loop(0, kb // 16, bg, 0)
        pltpu.sync_copy(bo, bn_hbm.at[pl.ds(pl.multiple_of(bb, 8), kb)])

    return k(perm, btab, s_ep, d_ep)


def tc_dinv(deg2, gate):
    npad = deg2.shape[1]

    def body(deg_ref, gate_ref, dinv_ref, s_ref):
        deg = deg_ref[0, :] + deg_ref[1, :] + 1.0
        dinv = lax.rsqrt(deg)
        dinv_ref[0, :] = dinv
        s_ref[0, :] = dinv * gate_ref[0, :]

    return pl.pallas_call(
        body,
        out_shape=(jax.ShapeDtypeStruct((1, npad), jnp.float32),
                   jax.ShapeDtypeStruct((1, npad), jnp.float32)),
    )(deg2, gate)


def tc_matmul_scale(h, w, s3):
    npad, din = h.shape
    dout = w.shape[1]
    nb = npad // BM

    def body(h_ref, w_ref, s_ref, y_ref):
        y = jnp.dot(h_ref[...], w_ref[...], preferred_element_type=jnp.float32)
        y_ref[...] = y * s_ref[0, 0, :][:, None]

    return pl.pallas_call(
        body,
        grid=(nb,),
        in_specs=[
            pl.BlockSpec((BM, din), lambda i: (i, 0)),
            pl.BlockSpec((din, dout), lambda i: (0, 0)),
            pl.BlockSpec((1, 1, BM), lambda i: (i, 0, 0)),
        ],
        out_specs=pl.BlockSpec((BM, dout), lambda i: (i, 0)),
        out_shape=jax.ShapeDtypeStruct((npad, dout), jnp.float32),
    )(h, w, s3)


def tc_epilogue(acc2, y, dinv3, b, p):
    npad, d = y.shape
    nb = npad // BM

    def body(acc_ref, y_ref, dinv_ref, b_ref, p_ref, h_ref, sc_ref):
        a = acc_ref[0] + acc_ref[1] + y_ref[...]
        h = a * dinv_ref[0, 0, :][:, None] + b_ref[0, :][None, :]
        h = jnp.maximum(h, 0.0)
        h_ref[...] = h
        pv = p_ref[0, :]
        pn = pv * lax.rsqrt(jnp.sum(pv * pv))
        sc_ref[0, 0, :] = jnp.tanh(jnp.sum(h * pn[None, :], axis=1))

    return pl.pallas_call(
        body,
        grid=(nb,),
        in_specs=[
            pl.BlockSpec((2, BM, d), lambda i: (0, i, 0)),
            pl.BlockSpec((BM, d), lambda i: (i, 0)),
            pl.BlockSpec((1, 1, BM), lambda i: (i, 0, 0)),
            pl.BlockSpec((1, d), lambda i: (0, 0)),
            pl.BlockSpec((1, d), lambda i: (0, 0)),
        ],
        out_specs=(pl.BlockSpec((BM, d), lambda i: (i, 0)),
                   pl.BlockSpec((1, 1, BM), lambda i: (i, 0, 0))),
        out_shape=(jax.ShapeDtypeStruct((npad, d), jnp.float32),
                   jax.ShapeDtypeStruct((nb, 1, BM), jnp.float32)),
    )(acc2, y, dinv3, b, p)


def tc_readout(h, batch_row, batch_col):
    kpad, d = h.shape

    def body(h_ref, br_ref, bc_ref, z_ref):
        g = pl.program_id(0)
        bmf = (br_ref[...] == g).astype(jnp.float32)
        cnt = jnp.sum(bmf)
        gsum = jnp.dot(bmf, h_ref[...], preferred_element_type=jnp.float32)
        hm = jnp.where(bc_ref[...] == g, h_ref[...], NEG)
        gmx = jnp.max(hm, axis=0, keepdims=True)
        gmx = jnp.where(cnt > 0.0, gmx, 0.0)
        gap = gsum / jnp.maximum(cnt, 1.0)
        z_ref[0] = jnp.concatenate([gmx, gap], axis=1)

    return pl.pallas_call(
        body,
        grid=(NUM_GRAPHS,),
        in_specs=[
            pl.BlockSpec((kpad, d), lambda g: (0, 0)),
            pl.BlockSpec((1, kpad), lambda g: (0, 0)),
            pl.BlockSpec((kpad, 1), lambda g: (0, 0)),
        ],
        out_specs=pl.BlockSpec((1, 1, 2 * d), lambda g: (g, 0, 0)),
        out_shape=jax.ShapeDtypeStruct((NUM_GRAPHS, 1, 2 * d), jnp.float32),
    )(h, batch_row, batch_col)


def tc_mlp(z, wm1, bm1, wm2, bm2):
    g = z.shape[0]
    dc = wm2.shape[1]

    def body(z_ref, w1_ref, b1_ref, w2_ref, b2_ref, o_ref):
        a = jnp.dot(z_ref[...], w1_ref[...],
                    preferred_element_type=jnp.float32) + b1_ref[0, :][None, :]
        a = jnp.maximum(a, 0.0)
        o = jnp.dot(a, w2_ref[...],
                    preferred_element_type=jnp.float32) + b2_ref[0, :][None, :]
        m = jnp.max(o, axis=1, keepdims=True)
        lse = m + jnp.log(jnp.sum(jnp.exp(o - m), axis=1, keepdims=True))
        o_ref[...] = o - lse

    return pl.pallas_call(
        body,
        out_shape=jax.ShapeDtypeStruct((g, dc), jnp.float32),
    )(z, wm1, bm1, wm2, bm2)


def _edges_half(idx, fill):
    pad = jnp.full((EP - idx.shape[0],), fill, jnp.int32)
    return jnp.concatenate([idx.astype(jnp.int32), pad])


def _unravel2(a, np_acc):
    return a.reshape(NC, NS, np_acc // 256, 16).transpose(0, 2, 1, 3).reshape(
        NC, np_acc)


def _unravel3(a, np_acc, d):
    return a.reshape(NC, NS, np_acc // 256, 16, d).transpose(
        0, 2, 1, 3, 4).reshape(NC, np_acc, d)


def gcn_layer(h, gate, sh, dh, w, b, p, n, npad, np_acc):
    dout = w.shape[1]
    rows_pt = np_acc // NS
    deg_f, slist, llist, counts = sc_partition(sh, dh, np_acc, npad)
    deg_il = deg_f.reshape(NC, NS, rows_pt, 16)[..., 0]
    deg2 = _unravel2(deg_il, np_acc)[:, :npad]
    dinv, s_mm = tc_dinv(deg2, gate.reshape(1, npad))
    nb = npad // BM
    y = tc_matmul_scale(h, w, s_mm.reshape(nb, 1, BM))
    acc_f = sc_gather_accum(y, slist, llist, counts, np_acc, dout,
                            _chunk_for(dout))
    acc2 = _unravel3(acc_f.reshape(NC, NS, rows_pt, dout), np_acc, dout)
    hn, sc3 = tc_epilogue(acc2, y, dinv.reshape(nb, 1, BM),
                          b.reshape(1, -1), p.reshape(1, -1))
    score = sc3.reshape(npad)
    score = jnp.where(jnp.arange(npad) < n, score, NEG)
    return hn, score


def kernel(x, edge_index, batch, epoch, W1, b1, p1, W2, b2, p2, W3, b3,
           Wm1, bm1, Wm2, bm2):
    del epoch
    n1p, k1p, k2p = _pad_rows(N0), _pad_rows(K1), _pad_rows(K2)
    a1, a2, a3 = _pad_acc(N0), _pad_acc(K1), _pad_acc(K2)

    src = edge_index[0].astype(jnp.int32)
    dst = edge_index[1].astype(jnp.int32)
    batch = batch.astype(jnp.int32)

    xp = jnp.pad(x, ((0, n1p - N0), (0, 0)))
    g1 = jnp.where(jnp.arange(n1p) < N0, 1.0, 0.0).astype(jnp.float32)
    batch1 = jnp.pad(batch, (0, n1p - N0), constant_values=NUM_GRAPHS)
    s1 = _edges_half(src, 0)
    d1 = _edges_half(dst, n1p)

    h1, sc1 = gcn_layer(xp, g1, s1, d1, W1, b1, p1, N0, n1p, a1)

    vals1, perm1 = lax.top_k(sc1, K1)
    permp1 = jnp.pad(perm1, (0, k1p - K1), constant_values=N0)
    gate2 = jnp.pad(vals1, (0, k1p - K1))
    s2, d2, batch2 = sc_remap(permp1, batch1, s1, d1, n1p, k1p, k1p)
    h1p = h1[permp1]

    h2, sc2 = gcn_layer(h1p, gate2, s2, d2, W2, b2, p2, K1, k1p, a2)

    vals2, perm2 = lax.top_k(sc2, K2)
    permp2 = jnp.pad(perm2, (0, k2p - K2), constant_values=K1)
    gate3 = jnp.pad(vals2, (0, k2p - K2))
    s3, d3, batch3 = sc_remap(permp2, batch2, s2, d2, k1p, k2p, k2p)
    h2p = h2[permp2]

    h3, _ = gcn_layer(h2p, gate3, s3, d3, W3, b3,
                      jnp.ones((W3.shape[1],), jnp.float32), K2, k2p, a3)

    z = tc_readout(h3, batch3.reshape(1, k2p),
                   batch3.reshape(k2p, 1)).reshape(NUM_GRAPHS, -1)
    return tc_mlp(z, Wm1, bm1.reshape(1, -1), Wm2, bm2.reshape(1, -1))


# speedup vs baseline: 21.1275x; 1.1115x over previous
import functools
import math

import jax
import jax.numpy as jnp
from jax import lax
from jax.experimental import pallas as pl
from jax.experimental.pallas import tpu as pltpu
from jax.experimental.pallas import tpu_sc as plsc

N0 = 10000
NUM_GRAPHS = 64
E0 = 320000
K1 = int(math.ceil(0.5 * N0))
K2 = int(math.ceil(0.5 * K1))

NC = 2
NS = 16
NW = NC * NS
EP = 327680
BM = 256

NEG = -3.0e38


def _pad_rows(n):
    return ((n + BM - 1) // BM) * BM


def _pad_acc(n):
    return _pad_rows(n) + 256


def _chunk_for(d):
    return {128: 128, 256: 64, 512: 32}[d]


def _sc_mesh():
    return plsc.VectorSubcoreMesh(core_axis_name="c", subcore_axis_name="s",
                                  num_cores=NC, num_subcores=NS)


CF = 8192
NCF = EP // (NC * CF)
CAP = 167936
CBH = CF // 2 + 64
CBN = 2 * CBH


def sc_partition(sh, dh, np_acc, npad):
    rows_pt = np_acc // NS
    locpad = rows_pt - 16

    @functools.partial(
        pl.kernel,
        mesh=_sc_mesh(),
        compiler_params=pltpu.CompilerParams(needs_layout_passes=False),
        out_type=(jax.ShapeDtypeStruct((NW * rows_pt * 16,), jnp.float32),
                  jax.ShapeDtypeStruct((NW * CAP,), jnp.int32),
                  jax.ShapeDtypeStruct((NW * CAP,), jnp.int32),
                  jax.ShapeDtypeStruct((NW * 8,), jnp.int32)),
        scratch_types=[
            pltpu.VMEM((CF,), jnp.int32),
            pltpu.VMEM((CF,), jnp.int32),
            pltpu.VMEM((CBN,), jnp.int32),
            pltpu.VMEM((CBN,), jnp.int32),
            pltpu.VMEM((rows_pt * 16,), jnp.float32),
            pltpu.VMEM((16,), jnp.int32),
        ],
    )
    def k(sh_hbm, dh_hbm, deg_hbm, sl_hbm, ll_hbm, cnt_hbm,
          sv, dv, cbs, cbl, degv, cntv):
        c = lax.axis_index("c")
        s = lax.axis_index("s")
        w = c * NS + s
        base = w * CAP

        z16 = jnp.zeros((16,), jnp.float32)
        one16 = (lax.iota(jnp.int32, 16) == 0).astype(jnp.float32)
        iota16 = lax.iota(jnp.int32, 16)
        lp16 = jnp.full((16,), locpad, jnp.int32)
        z16i = jnp.zeros((16,), jnp.int32)

        def zdeg(i, _):
            degv[pl.ds(i * 16, 16)] = z16
            return 0
        lax.fori_loop(0, rows_pt, zdeg, 0)

        eph = NCF * CF

        ng2 = CF // 32

        def chunk(i, flushed):
            eo = pl.multiple_of(c * eph + i * CF, 8)
            pltpu.sync_copy(sh_hbm.at[pl.ds(eo, CF)], sv)
            pltpu.sync_copy(dh_hbm.at[pl.ds(eo, CF)], dv)

            def filt16(j, off, obase):
                s16 = sv[pl.ds(j * 16, 16)]
                d16 = dv[pl.ds(j * 16, 16)]
                owner = lax.shift_right_logical(d16, 4) & 15
                m = (owner == s) & (d16 < npad)
                loc = (lax.shift_left(lax.shift_right_logical(d16, 8), 4)
                       | (d16 & 15))
                pc = jnp.cumsum(m.astype(jnp.int32))
                pos = obase + off + pc - 1
                plsc.store_scatter(cbs, (pos,), s16, mask=m)
                plsc.store_scatter(cbl, (pos,), loc, mask=m)
                return off + pc[15]

            def grp(j, offs):
                oa, ob = offs
                oa = filt16(j, oa, 0)
                ob = filt16(ng2 + j, ob, CBH)
                return (oa, ob)
            cnt_a, cnt_b = lax.fori_loop(0, ng2, grp, (0, 0))

            plsc.store_scatter(cbs, (cnt_a + iota16,), z16i)
            plsc.store_scatter(cbl, (cnt_a + iota16,), lp16)
            plsc.store_scatter(cbs, (CBH + cnt_b + iota16,), z16i)
            plsc.store_scatter(cbl, (CBH + cnt_b + iota16,), lp16)

            na = (cnt_a + 15) // 16
            nb = (cnt_b + 15) // 16
            lppad = locpad * 16

            def dinc(g, _):
                lvec = jnp.where(g < na, cbl[pl.ds(g * 16, 16)] * 16, lppad)
                lvec2 = jnp.where(g < nb,
                                  cbl[pl.ds(CBH + g * 16, 16)] * 16, lppad)
                for kk in range(16):
                    lo = pl.multiple_of(lvec[kk], 16)
                    plsc.addupdate(degv.at[pl.ds(lo, 16)], one16)
                    lo2 = pl.multiple_of(lvec2[kk], 16)
                    plsc.addupdate(degv.at[pl.ds(lo2, 16)], one16)
                return 0
            lax.fori_loop(0, jnp.maximum(na, nb), dinc, 0)

            ca8 = (cnt_a + 7) & (-8)
            cb8 = (cnt_b + 7) & (-8)
            fo = pl.multiple_of(base + flushed, 8)
            pltpu.sync_copy(cbs.at[pl.ds(0, CBH)], sl_hbm.at[pl.ds(fo, CBH)])
            pltpu.sync_copy(cbl.at[pl.ds(0, CBH)], ll_hbm.at[pl.ds(fo, CBH)])
            fo2 = pl.multiple_of(base + flushed + ca8, 8)
            pltpu.sync_copy(cbs.at[pl.ds(CBH, CBH)],
                            sl_hbm.at[pl.ds(fo2, CBH)])
            pltpu.sync_copy(cbl.at[pl.ds(CBH, CBH)],
                            ll_hbm.at[pl.ds(fo2, CBH)])
            return flushed + ca8 + cb8
        total = lax.fori_loop(0, NCF, chunk, 0)

        cbs[pl.ds(0, 16)] = z16i
        cbl[pl.ds(0, 16)] = lp16

        def fpad(i, _):
            po = pl.multiple_of(base + total + i * 16, 8)
            pltpu.sync_copy(cbs.at[pl.ds(0, 16)], sl_hbm.at[pl.ds(po, 16)])
            pltpu.sync_copy(cbl.at[pl.ds(0, 16)], ll_hbm.at[pl.ds(po, 16)])
            return 0
        lax.fori_loop(0, 8, fpad, 0)

        cntv[...] = jnp.full((16,), 0, jnp.int32) + total
        pltpu.sync_copy(cntv.at[pl.ds(0, 8)],
                        cnt_hbm.at[pl.ds(pl.multiple_of(w * 8, 8), 8)])
        do = pl.multiple_of(w * rows_pt * 16, 8)
        pltpu.sync_copy(degv, deg_hbm.at[pl.ds(do, rows_pt * 16)])

    return k(sh, dh)


def sc_gather_accum(y, slist, llist, counts, np_acc, d, ch):
    rows_pt = np_acc // NS

    @functools.partial(
        pl.kernel,
        mesh=_sc_mesh(),
        compiler_params=pltpu.CompilerParams(needs_layout_passes=False),
        out_type=jax.ShapeDtypeStruct((NW * rows_pt * d,), jnp.float32),
        scratch_types=[
            pltpu.VMEM((ch,), jnp.int32),
            pltpu.VMEM((ch,), jnp.int32),
            pltpu.VMEM((ch,), jnp.int32),
            pltpu.VMEM((ch,), jnp.int32),
            pltpu.VMEM((ch, d), jnp.float32),
            pltpu.VMEM((ch, d), jnp.float32),
            pltpu.VMEM((rows_pt * d,), jnp.float32),
            pltpu.VMEM((16,), jnp.int32),
            pltpu.SemaphoreType.DMA,
            pltpu.SemaphoreType.DMA,
            pltpu.SemaphoreType.DMA,
            pltpu.SemaphoreType.DMA,
            pltpu.SemaphoreType.DMA,
            pltpu.SemaphoreType.DMA,
        ],
    )
    def k(y_hbm, sl_hbm, ll_hbm, cnt_hbm, out_hbm, sbuf0, sbuf1, lbuf0,
          lbuf1, rows0, rows1, acc, cntv, si0, si1, li0, li1, sg0, sg1):
        c = lax.axis_index("c")
        s = lax.axis_index("s")
        w = c * NS + s
        base = w * CAP
        sbufs, lbufs = [sbuf0, sbuf1], [lbuf0, lbuf1]
        rowss, sis, lis, sgs = [rows0, rows1], [si0, si1], [li0, li1], \
            [sg0, sg1]

        z16 = jnp.zeros((16,), jnp.float32)

        def zacc(i, _):
            rb = pl.multiple_of(i * d, 8)
            for kk in range(d // 16):
                acc[pl.ds(rb + kk * 16, 16)] = z16
            return 0
        lax.fori_loop(0, rows_pt, zacc, 0)

        pltpu.sync_copy(cnt_hbm.at[pl.ds(pl.multiple_of(w * 8, 8), 8)],
                        cntv.at[pl.ds(0, 8)])
        cnt = cntv[...][0]
        nch = (cnt + (ch - 1)) // ch

        def _off(j):
            return pl.multiple_of(base + j * ch, 8)

        @pl.when(nch > 0)
        def _prologue():
            pltpu.sync_copy(sl_hbm.at[pl.ds(_off(0), ch)], sbufs[0])
            pltpu.sync_copy(ll_hbm.at[pl.ds(_off(0), ch)], lbufs[0])
            pltpu.async_copy(y_hbm.at[sbufs[0]], rowss[0], sgs[0])

        def pair(i, _):
            for b in range(2):
                kk_ = 2 * i + b

                @pl.when(kk_ < nch)
                def _(b=b, kk_=kk_):
                    nxt = kk_ + 1

                    @pl.when(nxt < nch)
                    def _():
                        pltpu.async_copy(sl_hbm.at[pl.ds(_off(nxt), ch)],
                                         sbufs[1 - b], sis[1 - b])
                        pltpu.async_copy(ll_hbm.at[pl.ds(_off(nxt), ch)],
                                         lbufs[1 - b], lis[1 - b])

                    pltpu.make_async_copy(y_hbm.at[sbufs[b]], rowss[b],
                                          sgs[b]).wait()

                    @pl.when(nxt < nch)
                    def _():
                        pltpu.make_async_copy(
                            sl_hbm.at[pl.ds(_off(nxt), ch)], sbufs[1 - b],
                            sis[1 - b]).wait()
                        pltpu.make_async_copy(
                            ll_hbm.at[pl.ds(_off(nxt), ch)], lbufs[1 - b],
                            lis[1 - b]).wait()
                        pltpu.async_copy(y_hbm.at[sbufs[1 - b]],
                                         rowss[1 - b], sgs[1 - b])

                    def grp(g, _):
                        lvec = lbufs[b][pl.ds(g * 16, 16)] * d
                        for k2 in range(16):
                            lb = pl.multiple_of(lvec[k2], 8)
                            for kk in range(d // 16):
                                plsc.addupdate(
                                    acc.at[pl.ds(lb + kk * 16, 16)],
                                    rowss[b][g * 16 + k2,
                                             pl.ds(kk * 16, 16)])
                        return 0
                    lax.fori_loop(0, ch // 16, grp, 0)
            return 0
        lax.fori_loop(0, (nch + 1) // 2, pair, 0)

        oo = pl.multiple_of(w * rows_pt * d, 8)
        pltpu.sync_copy(acc, out_hbm.at[pl.ds(oo, rows_pt * d)])

    return k(y, slist, llist, counts)


CR = 2048
NER = EP // NW
NCR = NER // CR


def sc_remap(perm, btab, s_ep, d_ep, npad_prev, npad_next, kpad):
    kb = kpad // NW

    @functools.partial(
        pl.kernel,
        mesh=_sc_mesh(),
        compiler_params=pltpu.CompilerParams(needs_layout_passes=False),
        out_type=(jax.ShapeDtypeStruct((EP,), jnp.int32),
                  jax.ShapeDtypeStruct((EP,), jnp.int32),
                  jax.ShapeDtypeStruct((kpad,), jnp.int32)),
        scratch_types=[
            pltpu.VMEM((kpad,), jnp.int32),
            pltpu.VMEM((npad_prev,), jnp.int32),
            pltpu.VMEM((npad_prev,), jnp.int32),
            pltpu.VMEM((CR,), jnp.int32),
            pltpu.VMEM((CR,), jnp.int32),
            pltpu.VMEM((CR,), jnp.int32),
            pltpu.VMEM((CR,), jnp.int32),
            pltpu.VMEM((kb,), jnp.int32),
        ],
    )
    def k(perm_hbm, btab_hbm, s_hbm, d_hbm, sn_hbm, dn_hbm, bn_hbm,
          pb, nidx, bt, sv, dv, so, do, bo):
        c = lax.axis_index("c")
        s = lax.axis_index("s")
        w = c * NS + s

        iota16 = lax.iota(jnp.int32, 16)
        neg16 = jnp.full((16,), -1, jnp.int32)

        pltpu.sync_copy(perm_hbm, pb)
        pltpu.sync_copy(btab_hbm, bt)

        def zn(i, _):
            nidx[pl.ds(i * 16, 16)] = neg16
            return 0
        lax.fori_loop(0, npad_prev // 16, zn, 0)

        def sca(g, _):
            pv = pb[pl.ds(g * 16, 16)]
            plsc.store_scatter(nidx, (pv,), g * 16 + iota16)
            return 0
        lax.fori_loop(0, kpad // 16, sca, 0)

        def chunk(i, _):
            eo = pl.multiple_of(w * NER + i * CR, 8)
            pltpu.sync_copy(s_hbm.at[pl.ds(eo, CR)], sv)
            pltpu.sync_copy(d_hbm.at[pl.ds(eo, CR)], dv)

            def grp(j, _):
                s16 = sv[pl.ds(j * 16, 16)]
                d16 = dv[pl.ds(j * 16, 16)]
                valid = d16 < npad_prev
                dcl = jnp.where(valid, d16, 0)
                sg = plsc.load_gather(nidx, (s16,))
                dg = plsc.load_gather(nidx, (dcl,))
                m = valid & (sg >= 0) & (dg >= 0)
                so[pl.ds(j * 16, 16)] = jnp.where(m, sg, 0)
                do[pl.ds(j * 16, 16)] = jnp.where(m, dg, npad_next)
                return 0
            lax.fori_loop(0, CR // 16, grp, 0)

            pltpu.sync_copy(so, sn_hbm.at[pl.ds(eo, CR)])
            pltpu.sync_copy(do, dn_hbm.at[pl.ds(eo, CR)])
            return 0
        lax.fori_loop(0, NCR, chunk, 0)

        bb = w * kb

        def bg(g, _):
            pv = pb[pl.ds(bb + g * 16, 16)]
            bo[pl.ds(g * 16, 16)] = plsc.load_gather(bt, (pv,))
            return 0
        lax.fori_loop(0, kb // 16, bg, 0)
        pltpu.sync_copy(bo, bn_hbm.at[pl.ds(pl.multiple_of(bb, 8), kb)])

    return k(perm, btab, s_ep, d_ep)


def tc_dinv(deg2, gate):
    npad = deg2.shape[1]

    def body(deg_ref, gate_ref, dinv_ref, s_ref):
        deg = deg_ref[0, :] + deg_ref[1, :] + 1.0
        dinv = lax.rsqrt(deg)
        dinv_ref[0, :] = dinv
        s_ref[0, :] = dinv * gate_ref[0, :]

    return pl.pallas_call(
        body,
        out_shape=(jax.ShapeDtypeStruct((1, npad), jnp.float32),
                   jax.ShapeDtypeStruct((1, npad), jnp.float32)),
    )(deg2, gate)


def tc_matmul_scale(h, w, s3):
    npad, din = h.shape
    dout = w.shape[1]
    nb = npad // BM

    def body(h_ref, w_ref, s_ref, y_ref):
        y = jnp.dot(h_ref[...], w_ref[...], preferred_element_type=jnp.float32)
        y_ref[...] = y * s_ref[0, 0, :][:, None]

    return pl.pallas_call(
        body,
        grid=(nb,),
        in_specs=[
            pl.BlockSpec((BM, din), lambda i: (i, 0)),
            pl.BlockSpec((din, dout), lambda i: (0, 0)),
            pl.BlockSpec((1, 1, BM), lambda i: (i, 0, 0)),
        ],
        out_specs=pl.BlockSpec((BM, dout), lambda i: (i, 0)),
        out_shape=jax.ShapeDtypeStruct((npad, dout), jnp.float32),
    )(h, w, s3)


def tc_epilogue(acc2, y, dinv3, b, p):
    npad, d = y.shape
    nb = npad // BM

    def body(acc_ref, y_ref, dinv_ref, b_ref, p_ref, h_ref, sc_ref):
        a = acc_ref[0] + acc_ref[1] + y_ref[...]
        h = a * dinv_ref[0, 0, :][:, None] + b_ref[0, :][None, :]
        h = jnp.maximum(h, 0.0)
        h_ref[...] = h
        pv = p_ref[0, :]
        pn = pv * lax.rsqrt(jnp.sum(pv * pv))
        sc_ref[0, 0, :] = jnp.tanh(jnp.sum(h * pn[None, :], axis=1))

    return pl.pallas_call(
        body,
        grid=(nb,),
        in_specs=[
            pl.BlockSpec((2, BM, d), lambda i: (0, i, 0)),
            pl.BlockSpec((BM, d), lambda i: (i, 0)),
            pl.BlockSpec((1, 1, BM), lambda i: (i, 0, 0)),
            pl.BlockSpec((1, d), lambda i: (0, 0)),
            pl.BlockSpec((1, d), lambda i: (0, 0)),
        ],
        out_specs=(pl.BlockSpec((BM, d), lambda i: (i, 0)),
                   pl.BlockSpec((1, 1, BM), lambda i: (i, 0, 0))),
        out_shape=(jax.ShapeDtypeStruct((npad, d), jnp.float32),
                   jax.ShapeDtypeStruct((nb, 1, BM), jnp.float32)),
    )(acc2, y, dinv3, b, p)


def tc_readout(h, batch_row, batch_col):
    kpad, d = h.shape

    def body(h_ref, br_ref, bc_ref, z_ref):
        g = pl.program_id(0)
        bmf = (br_ref[...] == g).astype(jnp.float32)
        cnt = jnp.sum(bmf)
        gsum = jnp.dot(bmf, h_ref[...], preferred_element_type=jnp.float32)
        hm = jnp.where(bc_ref[...] == g, h_ref[...], NEG)
        gmx = jnp.max(hm, axis=0, keepdims=True)
        gmx = jnp.where(cnt > 0.0, gmx, 0.0)
        gap = gsum / jnp.maximum(cnt, 1.0)
        z_ref[0] = jnp.concatenate([gmx, gap], axis=1)

    return pl.pallas_call(
        body,
        grid=(NUM_GRAPHS,),
        in_specs=[
            pl.BlockSpec((kpad, d), lambda g: (0, 0)),
            pl.BlockSpec((1, kpad), lambda g: (0, 0)),
            pl.BlockSpec((kpad, 1), lambda g: (0, 0)),
        ],
        out_specs=pl.BlockSpec((1, 1, 2 * d), lambda g: (g, 0, 0)),
        out_shape=jax.ShapeDtypeStruct((NUM_GRAPHS, 1, 2 * d), jnp.float32),
    )(h, batch_row, batch_col)


def tc_mlp(z, wm1, bm1, wm2, bm2):
    g = z.shape[0]
    dc = wm2.shape[1]

    def body(z_ref, w1_ref, b1_ref, w2_ref, b2_ref, o_ref):
        a = jnp.dot(z_ref[...], w1_ref[...],
                    preferred_element_type=jnp.float32) + b1_ref[0, :][None, :]
        a = jnp.maximum(a, 0.0)
        o = jnp.dot(a, w2_ref[...],
                    preferred_element_type=jnp.float32) + b2_ref[0, :][None, :]
        m = jnp.max(o, axis=1, keepdims=True)
        lse = m + jnp.log(jnp.sum(jnp.exp(o - m), axis=1, keepdims=True))
        o_ref[...] = o - lse

    return pl.pallas_call(
        body,
        out_shape=jax.ShapeDtypeStruct((g, dc), jnp.float32),
    )(z, wm1, bm1, wm2, bm2)


def _edges_half(idx, fill):
    pad = jnp.full((EP - idx.shape[0],), fill, jnp.int32)
    return jnp.concatenate([idx.astype(jnp.int32), pad])


def _unravel2(a, np_acc):
    return a.reshape(NC, NS, np_acc // 256, 16).transpose(0, 2, 1, 3).reshape(
        NC, np_acc)


def _unravel3(a, np_acc, d):
    return a.reshape(NC, NS, np_acc // 256, 16, d).transpose(
        0, 2, 1, 3, 4).reshape(NC, np_acc, d)


def gcn_layer(h, gate, sh, dh, w, b, p, n, npad, np_acc):
    dout = w.shape[1]
    rows_pt = np_acc // NS
    deg_f, slist, llist, counts = sc_partition(sh, dh, np_acc, npad)
    deg_il = deg_f.reshape(NC, NS, rows_pt, 16)[..., 0]
    deg2 = _unravel2(deg_il, np_acc)[:, :npad]
    dinv, s_mm = tc_dinv(deg2, gate.reshape(1, npad))
    nb = npad // BM
    y = tc_matmul_scale(h, w, s_mm.reshape(nb, 1, BM))
    acc_f = sc_gather_accum(y, slist, llist, counts, np_acc, dout,
                            _chunk_for(dout))
    acc2 = _unravel3(acc_f.reshape(NC, NS, rows_pt, dout), np_acc, dout)
    hn, sc3 = tc_epilogue(acc2, y, dinv.reshape(nb, 1, BM),
                          b.reshape(1, -1), p.reshape(1, -1))
    score = sc3.reshape(npad)
    score = jnp.where(jnp.arange(npad) < n, score, NEG)
    return hn, score


def kernel(x, edge_index, batch, epoch, W1, b1, p1, W2, b2, p2, W3, b3,
           Wm1, bm1, Wm2, bm2):
    del epoch
    n1p, k1p, k2p = _pad_rows(N0), _pad_rows(K1), _pad_rows(K2)
    a1, a2, a3 = _pad_acc(N0), _pad_acc(K1), _pad_acc(K2)

    src = edge_index[0].astype(jnp.int32)
    dst = edge_index[1].astype(jnp.int32)
    batch = batch.astype(jnp.int32)

    xp = jnp.pad(x, ((0, n1p - N0), (0, 0)))
    g1 = jnp.where(jnp.arange(n1p) < N0, 1.0, 0.0).astype(jnp.float32)
    batch1 = jnp.pad(batch, (0, n1p - N0), constant_values=NUM_GRAPHS)
    s1 = _edges_half(src, 0)
    d1 = _edges_half(dst, n1p)

    h1, sc1 = gcn_layer(xp, g1, s1, d1, W1, b1, p1, N0, n1p, a1)

    vals1, perm1 = lax.top_k(sc1, K1)
    permp1 = jnp.pad(perm1, (0, k1p - K1), constant_values=N0)
    gate2 = jnp.pad(vals1, (0, k1p - K1))
    s2, d2, batch2 = sc_remap(permp1, batch1, s1, d1, n1p, k1p, k1p)
    h1p = h1[permp1]

    h2, sc2 = gcn_layer(h1p, gate2, s2, d2, W2, b2, p2, K1, k1p, a2)

    vals2, perm2 = lax.top_k(sc2, K2)
    permp2 = jnp.pad(perm2, (0, k2p - K2), constant_values=K1)
    gate3 = jnp.pad(vals2, (0, k2p - K2))
    s3, d3, batch3 = sc_remap(permp2, batch2, s2, d2, k1p, k2p, k2p)
    h2p = h2[permp2]

    h3, _ = gcn_layer(h2p, gate3, s3, d3, W3, b3,
                      jnp.ones((W3.shape[1],), jnp.float32), K2, k2p, a3)

    z = tc_readout(h3, batch3.reshape(1, k2p),
                   batch3.reshape(k2p, 1)).reshape(NUM_GRAPHS, -1)
    return tc_mlp(z, Wm1, bm1.reshape(1, -1), Wm2, bm2.reshape(1, -1))
